# Initial kernel scaffold; baseline (speedup 1.0000x reference)
import jax
import jax.numpy as jnp
from jax.experimental import pallas as pl


def kernel(local_prosody, codebooks, W1, b1, g1, bt1, W2, b2, g2, bt2, l1_indices, temperature):
    raise NotImplementedError("write your pallas kernel here")


# speedup vs baseline: 3.0217x; 3.0217x over previous
import functools

import jax
import jax.numpy as jnp
from jax.experimental import pallas as pl
from jax.experimental.pallas import tpu as pltpu

B, T = 8, 2048
D_MODEL = 1024
N_L2 = 1024
BD = 256
TB = 512
NT = T // TB


def _ln(x, g, b, eps=1e-5):
    m = jnp.mean(x, axis=-1, keepdims=True)
    v = jnp.mean((x - m) ** 2, axis=-1, keepdims=True)
    return (x - m) / jnp.sqrt(v + eps) * g + b


def _body(idx_ref, temp_ref, x_ref, cb_ref, W1_ref, b1_ref, g1_ref, bt1_ref,
          W2_ref, b2_ref, g2_ref, bt2_ref,
          hard_ref, soft_ref, emb_ref, emblow_ref):
    x = x_ref[0]
    cb = cb_ref[0]
    temp = temp_ref[0]

    h0 = jnp.dot(x, W1_ref[...], preferred_element_type=jnp.float32) + b1_ref[...]
    h = _ln(h0, g1_ref[...], bt1_ref[...])
    hn = h / jnp.maximum(jnp.sqrt(jnp.sum(h * h, axis=-1, keepdims=True)), 1e-12)

    cb_inv = 1.0 / jnp.maximum(
        jnp.sqrt(jnp.sum(cb * cb, axis=-1, keepdims=True)), 1e-12)
    cbn = cb * cb_inv

    logits = jnp.dot(hn, cbn.T, preferred_element_type=jnp.float32) / temp

    rowmax = jnp.max(logits, axis=-1, keepdims=True)
    e = jnp.exp(logits - rowmax)
    soft_ref[0] = e / jnp.sum(e, axis=-1, keepdims=True)

    kiota = jax.lax.broadcasted_iota(jnp.int32, logits.shape, 1)
    idx = jnp.min(jnp.where(logits == rowmax, kiota, N_L2), axis=-1,
                  keepdims=True)
    hard_ref[0, 0] = idx.T.astype(jnp.int32)

    onehot = (kiota == idx).astype(jnp.float32)
    emb_low = jnp.dot(onehot, cb, preferred_element_type=jnp.float32)
    emblow_ref[0] = emb_low

    e0 = jnp.dot(emb_low, W2_ref[...], preferred_element_type=jnp.float32) + b2_ref[...]
    emb_ref[0] = _ln(e0, g2_ref[...], bt2_ref[...])


@jax.jit
def _run(local_prosody, codebooks, W1, b1, g1, bt1, W2, b2, g2, bt2,
         l1_indices, temperature):
    grid_spec = pltpu.PrefetchScalarGridSpec(
        num_scalar_prefetch=1,
        grid=(B, NT),
        in_specs=[
            pl.BlockSpec(memory_space=pltpu.SMEM),
            pl.BlockSpec((1, TB, D_MODEL), lambda b, t, i: (b, t, 0)),
            pl.BlockSpec((1, N_L2, BD), lambda b, t, i: (i[b], 0, 0)),
            pl.BlockSpec((D_MODEL, BD), lambda b, t, i: (0, 0)),
            pl.BlockSpec((BD,), lambda b, t, i: (0,)),
            pl.BlockSpec((BD,), lambda b, t, i: (0,)),
            pl.BlockSpec((BD,), lambda b, t, i: (0,)),
            pl.BlockSpec((BD, D_MODEL), lambda b, t, i: (0, 0)),
            pl.BlockSpec((D_MODEL,), lambda b, t, i: (0,)),
            pl.BlockSpec((D_MODEL,), lambda b, t, i: (0,)),
            pl.BlockSpec((D_MODEL,), lambda b, t, i: (0,)),
        ],
        out_specs=[
            pl.BlockSpec((1, 1, 1, TB), lambda b, t, i: (b, t, 0, 0)),
            pl.BlockSpec((1, TB, N_L2), lambda b, t, i: (b, t, 0)),
            pl.BlockSpec((1, TB, D_MODEL), lambda b, t, i: (b, t, 0)),
            pl.BlockSpec((1, TB, BD), lambda b, t, i: (b, t, 0)),
        ],
    )
    hard4, soft, emb, emb_low = pl.pallas_call(
        _body,
        grid_spec=grid_spec,
        out_shape=[
            jax.ShapeDtypeStruct((B, NT, 1, TB), jnp.int32),
            jax.ShapeDtypeStruct((B, T, N_L2), jnp.float32),
            jax.ShapeDtypeStruct((B, T, D_MODEL), jnp.float32),
            jax.ShapeDtypeStruct((B, T, BD), jnp.float32),
        ],
    )(l1_indices.astype(jnp.int32),
      jnp.reshape(jnp.asarray(temperature, jnp.float32), (1,)),
      local_prosody, codebooks, W1, b1, g1, bt1, W2, b2, g2, bt2)
    return hard4.reshape(B, T), soft, emb, emb_low


def kernel(local_prosody, codebooks, W1, b1, g1, bt1, W2, b2, g2, bt2,
           l1_indices, temperature):
    return _run(local_prosody, codebooks, W1, b1, g1, bt1, W2, b2, g2, bt2,
                l1_indices, temperature)
